# baseline (speedup 1.0000x reference)
import functools

import jax
import jax.numpy as jnp
from jax import lax
from jax.experimental import pallas as pl
from jax.experimental.pallas import tpu as pltpu
from jax.experimental.pallas import tpu_sc as plsc

_HIDDEN = 1024
_BATCH = 4
_SEQ = 8192
_B = _BATCH * _SEQ

_NC, _NS = 2, 16
_NW = _NC * _NS
_BPW = _B // _NW


@jax.jit
def _seg_embed(ids_flat, table):
    mesh = plsc.VectorSubcoreMesh(core_axis_name="c", subcore_axis_name="s")

    @functools.partial(
        pl.kernel,
        out_type=jax.ShapeDtypeStruct((_B, _HIDDEN), jnp.float32),
        mesh=mesh,
        compiler_params=pltpu.CompilerParams(needs_layout_passes=False),
        scratch_types=[
            pltpu.VMEM((_BPW,), jnp.int32),
            pltpu.VMEM((2, _HIDDEN), jnp.float32),
            pltpu.SemaphoreType.DMA,
            pltpu.SemaphoreType.DMA,
        ],
    )
    def k(ids_hbm, table_hbm, out_hbm, idx_v, tbl, semw, semin):
        s = lax.axis_index("s")
        c_ax = lax.axis_index("c")
        wid = s * _NC + c_ax
        base = wid * _BPW

        ids_cp = pltpu.make_async_copy(
            ids_hbm.at[pl.ds(base, _BPW)], idx_v, semin)
        ids_cp.start()
        pltpu.sync_copy(table_hbm, tbl)
        ids_cp.wait()

        _VL = 16

        @pl.loop(0, _BPW // _VL)
        def _(i):
            v = idx_v[pl.ds(i * _VL, _VL)]
            for j in range(_VL):
                row = jax.lax.squeeze(jax.lax.slice(v, (j,), (j + 1,)), (0,))
                pltpu.async_copy(
                    tbl.at[pl.ds(row, 1)],
                    out_hbm.at[pl.ds(base + i * _VL + j, 1)], semw)

        @pl.loop(0, _BPW)
        def _(p):
            pltpu.make_async_copy(tbl.at[pl.ds(0, 1)],
                                  out_hbm.at[pl.ds(base, 1)], semw).wait()

    return k(ids_flat, table)


def kernel(segment_ids, table):
    ids_flat = segment_ids.reshape(-1).astype(jnp.int32)
    out = _seg_embed(ids_flat, table)
    return out.reshape(_BATCH, _SEQ, _HIDDEN)


# speedup vs baseline: 1.0183x; 1.0183x over previous
import functools

import jax
import jax.numpy as jnp
from jax import lax
from jax.experimental import pallas as pl
from jax.experimental.pallas import tpu as pltpu
from jax.experimental.pallas import tpu_sc as plsc

_HIDDEN = 1024
_BATCH = 4
_SEQ = 8192
_B = _BATCH * _SEQ

_NC, _NS = 2, 16
_NW = _NC * _NS
_BPW = _B // _NW
_VL = 16
_GW = 8
_GSH = 3
_NG = _BPW // _GW


@jax.jit
def _seg_embed(ids_flat, table):
    mesh = plsc.VectorSubcoreMesh(core_axis_name="c", subcore_axis_name="s")

    @functools.partial(
        pl.kernel,
        out_type=jax.ShapeDtypeStruct((_B, _HIDDEN), jnp.float32),
        mesh=mesh,
        compiler_params=pltpu.CompilerParams(needs_layout_passes=False),
        scratch_types=[
            pltpu.VMEM((_BPW,), jnp.int32),
            pltpu.VMEM((2, _HIDDEN), jnp.float32),
            pltpu.VMEM((_GW, _HIDDEN), jnp.float32),
            pltpu.VMEM((_GW, _HIDDEN), jnp.float32),
            pltpu.VMEM(((_NG + 1) * _GW,), jnp.int32),
            pltpu.VMEM(((_NG + 1) * _GW,), jnp.int32),
            pltpu.SemaphoreType.DMA,
            pltpu.SemaphoreType.DMA,
        ],
    )
    def k(ids_hbm, table_hbm, out_hbm, idx_v, tbl, rep0, rep1,
          pos0b, pos1b, semw, semin):
        s = lax.axis_index("s")
        c_ax = lax.axis_index("c")
        wid = s * _NC + c_ax
        base = wid * _BPW

        ids_cp = pltpu.make_async_copy(
            ids_hbm.at[pl.ds(base, _BPW)], idx_v, semin)
        ids_cp.start()
        pltpu.sync_copy(table_hbm, tbl)

        lanes = jax.lax.iota(jnp.int32, _VL)

        @pl.loop(0, _GW)
        def _(r):
            for j in range(_HIDDEN // _VL):
                sl = pl.ds(j * _VL, _VL)
                rep0[r, sl] = tbl[0, sl]
                rep1[r, sl] = tbl[1, sl]

        ids_cp.wait()

        def lane_gather(v, idx):
            return jax.lax.gather(
                v, idx[:, None],
                jax.lax.GatherDimensionNumbers(
                    offset_dims=(), collapsed_slice_dims=(0,),
                    start_index_map=(0,)),
                slice_sizes=(1,),
                mode=jax.lax.GatherScatterMode.PROMISE_IN_BOUNDS)

        def lane_cumsum(x):
            r = x
            for k in (1, 2, 4, 8):
                shifted = lane_gather(r, jnp.maximum(lanes - k, 0))
                r = r + jnp.where(lanes >= k, shifted, 0)
            return r

        def scalar(v):
            return jax.lax.squeeze(jax.lax.slice(v, (0,), (1,)), (0,))

        last = jnp.full((_VL,), _VL - 1, dtype=jnp.int32)
        zero_v = jnp.zeros((_VL,), jnp.int32)

        @pl.loop(0, _BPW // _VL, init_carry=(zero_v, zero_v))
        def _(i, carry):
            c0v, c1v = carry
            v = idx_v[pl.ds(i * _VL, _VL)]
            posv = (base + i * _VL) + lanes
            m0 = v == 0
            cs0 = lane_cumsum(jnp.where(m0, 1, 0).astype(jnp.int32))
            cs1 = (lanes + 1) - cs0
            r0 = c0v + cs0 - 1
            r1 = c1v + cs1 - 1
            plsc.store_scatter(pos0b, [r0], posv, mask=m0)
            plsc.store_scatter(pos1b, [r1], posv,
                               mask=jnp.logical_not(m0))
            n0v = c0v + lane_gather(cs0, last)
            n1v = c1v + lane_gather(cs1, last)

            @pl.loop(scalar(c0v) >> _GSH, scalar(n0v) >> _GSH)
            def _(g):
                pltpu.async_copy(
                    rep0, out_hbm.at[pos0b.at[pl.ds(g * _GW, _GW)]], semw)

            @pl.loop(scalar(c1v) >> _GSH, scalar(n1v) >> _GSH)
            def _(g):
                pltpu.async_copy(
                    rep1, out_hbm.at[pos1b.at[pl.ds(g * _GW, _GW)]], semw)

            return (n0v, n1v)

        c0v, c1v = _
        c0 = scalar(c0v)
        c1 = scalar(c1v)

        def pad(pos_ref, cnt):
            grp_base = jnp.full((_VL,), (cnt >> _GSH) << _GSH,
                                dtype=jnp.int32)
            first = plsc.load_gather(pos_ref, [grp_base])
            rem = cnt & (_GW - 1)
            for h in range(max(1, _GW // _VL)):
                col = lanes + h * _VL
                plsc.store_scatter(
                    pos_ref, [grp_base + col], first,
                    mask=jnp.logical_and(col >= rem, col < _GW))

        pad(pos0b, c0)
        pad(pos1b, c1)

        n0 = (c0 + _GW - 1) >> _GSH
        n1 = (c1 + _GW - 1) >> _GSH

        @pl.loop(c0 >> _GSH, n0)
        def _(g):
            pltpu.async_copy(
                rep0, out_hbm.at[pos0b.at[pl.ds(g * _GW, _GW)]], semw)

        @pl.loop(c1 >> _GSH, n1)
        def _(g):
            pltpu.async_copy(
                rep1, out_hbm.at[pos1b.at[pl.ds(g * _GW, _GW)]], semw)

        @pl.loop(0, n0 + n1)
        def _(g):
            pltpu.make_async_copy(
                rep0, out_hbm.at[pos0b.at[pl.ds(0, _GW)]], semw).wait()

    return k(ids_flat, table)


def kernel(segment_ids, table):
    ids_flat = segment_ids.reshape(-1).astype(jnp.int32)
    out = _seg_embed(ids_flat, table)
    return out.reshape(_BATCH, _SEQ, _HIDDEN)
